# Initial kernel scaffold; baseline (speedup 1.0000x reference)
import jax
import jax.numpy as jnp
from jax.experimental import pallas as pl


def kernel(x_enc, x_mark_enc, conv_w, conv_b, w_temp, b_temp, down_w, down_b, convs_w, convs_b, up_w, up_b, wq, bq, wk, bk, wv, bv, wo, bo, w1, b1, w2, b2):
    raise NotImplementedError("write your pallas kernel here")


# speedup vs baseline: 2.2607x; 2.2607x over previous
import math

import numpy as np
import jax
import jax.numpy as jnp
from jax.experimental import pallas as pl

L = 2048
S = 2720
P = 3072
D = 768
H = 12
DK = 64
EPS = 1e-5
BW = 144
KW0 = BW + 32
KWU = BW + 32 + 512


def _pe_np():
    position = np.arange(L, dtype=np.float32)[:, None]
    div = np.exp(np.arange(0, D, 2, dtype=np.float32) * -(math.log(10000.0) / D))
    pe = np.zeros((L, D), dtype=np.float32)
    pe[:, 0::2] = np.sin(position * div)
    pe[:, 1::2] = np.cos(position * div)
    return pe


_PE = _pe_np()


def _elu(x):
    return jnp.where(x > 0, x, jnp.exp(jnp.minimum(x, 0.0)) - 1.0)


def _ln_rows(x):
    m = jnp.mean(x, axis=-1, keepdims=True)
    v = jnp.mean((x - m) * (x - m), axis=-1, keepdims=True)
    return (x - m) * jax.lax.rsqrt(v + EPS)


def _embed_kernel(xemb_ref, wemb_ref, bemb_ref, pe_ref, dw_ref, db_ref,
                  cw0_ref, cb0_ref, cw1_ref, cb1_ref, cw2_ref, cb2_ref,
                  uw_ref, ub_ref, out_ref):
    seq = jnp.dot(xemb_ref[...], wemb_ref[...], preferred_element_type=jnp.float32)
    seq = seq + bemb_ref[...] + pe_ref[...]
    tmp = jnp.dot(seq, dw_ref[...], preferred_element_type=jnp.float32) + db_ref[...]
    o1 = _elu(jnp.dot(tmp.reshape(512, 512), cw0_ref[...],
                            preferred_element_type=jnp.float32) + cb0_ref[...])
    o2 = _elu(jnp.dot(o1.reshape(128, 512), cw1_ref[...],
                            preferred_element_type=jnp.float32) + cb1_ref[...])
    o3 = _elu(jnp.dot(o2.reshape(32, 512), cw2_ref[...],
                            preferred_element_type=jnp.float32) + cb2_ref[...])
    allin = jnp.concatenate([o1, o2, o3], axis=0)
    allin = jnp.dot(allin, uw_ref[...], preferred_element_type=jnp.float32) + ub_ref[...]
    full = jnp.concatenate([seq, allin], axis=0)
    out_ref[0:S, :] = _ln_rows(full)
    out_ref[S:P, :] = jnp.zeros((P - S, D), jnp.float32)


def _qkv_kernel(x_ref, wq_ref, wk_ref, wv_ref, bq_ref, bk_ref, bv_ref,
                q_ref, k_ref, v_ref):
    x = x_ref[...]
    q = jnp.dot(x, wq_ref[...], preferred_element_type=jnp.float32) + bq_ref[...]
    q_ref[...] = q * (1.0 / 8.0)
    k_ref[...] = jnp.dot(x, wk_ref[...], preferred_element_type=jnp.float32) + bk_ref[...]
    v_ref[...] = jnp.dot(x, wv_ref[...], preferred_element_type=jnp.float32) + bv_ref[...]


def _post_kernel(a_ref, res_ref, wo_ref, bo_ref, out_ref):
    y = jnp.dot(a_ref[...], wo_ref[...], preferred_element_type=jnp.float32)
    out_ref[...] = _ln_rows(y + bo_ref[...] + res_ref[...])


def _ffn_kernel(x_ref, w1_ref, b1_ref, w2_ref, b2_ref, out_ref):
    x = x_ref[...]
    h = jax.nn.gelu(jnp.dot(x, w1_ref[...], preferred_element_type=jnp.float32)
                    + b1_ref[...])
    y = jnp.dot(h, w2_ref[...], preferred_element_type=jnp.float32) + b2_ref[...] + x
    out_ref[...] = _ln_rows(y)


def _attn_heads(q, kk, vv, ok, o_ref):
    for h in range(H):
        sl = slice(h * DK, (h + 1) * DK)
        s = jax.lax.dot_general(q[:, sl], kk[:, sl], (((1,), (1,)), ((), ())),
                                preferred_element_type=jnp.float32)
        s = jnp.where(ok, s, -1e9)
        m = jnp.max(s, axis=-1, keepdims=True)
        e = jnp.exp(s - m)
        p = e / jnp.sum(e, axis=-1, keepdims=True)
        o_ref[:, sl] = jnp.dot(p, vv[:, sl], preferred_element_type=jnp.float32)


def _attn_l0_kernel(q_ref, k_ref, v_ref, o_ref):
    i = pl.program_id(0)
    qs = i * 128
    bs = pl.multiple_of(jnp.maximum(qs - 8, 0), 8)
    ps = pl.multiple_of(2048 + i * 32, 8)
    kk = jnp.concatenate([k_ref[pl.ds(bs, BW), :], k_ref[pl.ds(ps, 32), :]], axis=0)
    vv = jnp.concatenate([v_ref[pl.ds(bs, BW), :], v_ref[pl.ds(ps, 32), :]], axis=0)
    row = jax.lax.broadcasted_iota(jnp.int32, (128, KW0), 0)
    col = jax.lax.broadcasted_iota(jnp.int32, (128, KW0), 1)
    qi = qs + row
    absj = bs + col
    band_ok = (jnp.abs(absj - qi) <= 2) & (absj < 2048) & (col < BW)
    par_ok = (col >= BW) & (col - BW == row // 4)
    _attn_heads(q_ref[...], kk, vv, band_ok | par_ok, o_ref)


def _attn_up_kernel(q_ref, k_ref, v_ref, o_ref):
    i = pl.program_id(0)
    qs = 2048 + 128 * i
    ls = jnp.where(i < 4, 2048, jnp.where(i < 5, 2560, 2688))
    le = jnp.where(i < 4, 2560, jnp.where(i < 5, 2688, 2720))
    ps = jnp.where(i < 4, 2560 + 32 * i, 2688)
    hasp = i < 5
    cs = jnp.where(i < 4, 512 * i, jnp.where(i < 5, 2048, 2560))
    clen = jnp.where(i < 5, 512, 128)
    bs = pl.multiple_of(qs - 8, 8)
    ps = pl.multiple_of(ps, 8)
    cs = pl.multiple_of(cs, 8)
    kk = jnp.concatenate([k_ref[pl.ds(bs, BW), :], k_ref[pl.ds(ps, 32), :],
                          k_ref[pl.ds(cs, 512), :]], axis=0)
    vv = jnp.concatenate([v_ref[pl.ds(bs, BW), :], v_ref[pl.ds(ps, 32), :],
                          v_ref[pl.ds(cs, 512), :]], axis=0)
    row = jax.lax.broadcasted_iota(jnp.int32, (128, KWU), 0)
    col = jax.lax.broadcasted_iota(jnp.int32, (128, KWU), 1)
    qi = qs + row
    absj = bs + col
    band_ok = (jnp.abs(absj - qi) <= 2) & (absj >= ls) & (absj < le) & (col < BW)
    pj = col - BW
    par_ok = (col >= BW) & (col < BW + 32) & (pj == row // 4) & hasp
    cj = col - (BW + 32)
    chl_ok = (col >= BW + 32) & (cj >= 4 * row) & (cj < 4 * row + 4) & (cj < clen)
    _attn_heads(q_ref[...], kk, vv, band_ok | par_ok | chl_ok, o_ref)


def _gather_kernel(s0_ref, s1_ref, s2_ref, s3_ref, out_ref):
    out_ref[:, 0 * D:1 * D] = s0_ref[...]
    out_ref[:, 1 * D:2 * D] = jnp.repeat(s1_ref[...], 4, axis=0)
    out_ref[:, 2 * D:3 * D] = jnp.repeat(s2_ref[...], 16, axis=0)
    out_ref[:, 3 * D:4 * D] = jnp.repeat(s3_ref[...], 64, axis=0)


def _full_spec():
    return pl.BlockSpec((P, D), lambda i: (0, 0))


def kernel(x_enc, x_mark_enc, conv_w, conv_b, w_temp, b_temp, down_w, down_b,
           convs_w, convs_b, up_w, up_b, wq, bq, wk, bk, wv, bv, wo, bo,
           w1, b1, w2, b2):
    f32 = jnp.float32
    x = x_enc[0]
    xm = x_mark_enc[0]
    xcat = jnp.concatenate(
        [jnp.roll(x, 1, axis=0), x, jnp.roll(x, -1, axis=0), xm], axis=1)
    xemb = jnp.pad(xcat, ((0, 0), (0, 128 - 25)))
    wemb = jnp.pad(jnp.concatenate([conv_w.reshape(21, D), w_temp], axis=0),
                   ((0, 128 - 25), (0, 0)))
    bemb = (conv_b + b_temp)[None]
    pe = jnp.asarray(_PE)
    cw = [convs_w[i].reshape(512, 128) for i in range(3)]
    cb = [convs_b[i][None] for i in range(3)]

    seq = pl.pallas_call(
        _embed_kernel,
        out_shape=jax.ShapeDtypeStruct((P, D), f32),
    )(xemb, wemb, bemb, pe, down_w, down_b[None],
      cw[0], cb[0], cw[1], cb[1], cw[2], cb[2], up_w, up_b[None])

    row_spec = pl.BlockSpec((256, D), lambda i: (i, 0))
    w_spec = pl.BlockSpec((D, D), lambda i: (0, 0))
    b_spec = pl.BlockSpec((1, D), lambda i: (0, 0))

    for l in range(2):
        q, k, v = pl.pallas_call(
            _qkv_kernel,
            grid=(P // 256,),
            in_specs=[row_spec, w_spec, w_spec, w_spec, b_spec, b_spec, b_spec],
            out_specs=[row_spec, row_spec, row_spec],
            out_shape=[jax.ShapeDtypeStruct((P, D), f32)] * 3,
        )(seq, wq[l], wk[l], wv[l], bq[l][None], bk[l][None], bv[l][None])

        qblk = pl.BlockSpec((128, D), lambda i: (i, 0))
        o0 = pl.pallas_call(
            _attn_l0_kernel,
            grid=(16,),
            in_specs=[qblk, _full_spec(), _full_spec()],
            out_specs=qblk,
            out_shape=jax.ShapeDtypeStruct((2048, D), f32),
        )(q[:2048], k, v)
        qup = pl.BlockSpec((128, D), lambda i: (16 + i, 0))
        o1 = pl.pallas_call(
            _attn_up_kernel,
            grid=(6,),
            in_specs=[qup, _full_spec(), _full_spec()],
            out_specs=pl.BlockSpec((128, D), lambda i: (i, 0)),
            out_shape=jax.ShapeDtypeStruct((768, D), f32),
        )(q, k, v)
        attn = jnp.concatenate([o0, o1, jnp.zeros((P - 2816, D), f32)], axis=0)

        seq = pl.pallas_call(
            _post_kernel,
            grid=(P // 256,),
            in_specs=[row_spec, row_spec, w_spec, b_spec],
            out_specs=row_spec,
            out_shape=jax.ShapeDtypeStruct((P, D), f32),
        )(attn, seq, wo[l], bo[l][None])

        seq = pl.pallas_call(
            _ffn_kernel,
            grid=(P // 256,),
            in_specs=[row_spec,
                      pl.BlockSpec((D, 2048), lambda i: (0, 0)),
                      pl.BlockSpec((1, 2048), lambda i: (0, 0)),
                      pl.BlockSpec((2048, D), lambda i: (0, 0)),
                      b_spec],
            out_specs=row_spec,
            out_shape=jax.ShapeDtypeStruct((P, D), f32),
        )(seq, w1[l], b1[l][None], w2[l], b2[l][None])

    out = pl.pallas_call(
        _gather_kernel,
        grid=(4,),
        in_specs=[pl.BlockSpec((512, D), lambda i: (i, 0)),
                  pl.BlockSpec((128, D), lambda i: (16 + i, 0)),
                  pl.BlockSpec((32, D), lambda i: (80 + i, 0)),
                  pl.BlockSpec((8, D), lambda i: (336 + i, 0))],
        out_specs=pl.BlockSpec((512, 4 * D), lambda i: (i, 0)),
        out_shape=jax.ShapeDtypeStruct((L, 4 * D), f32),
    )(seq, seq, seq, seq)
    return out[None]


# speedup vs baseline: 2.5609x; 1.1328x over previous
import math

import numpy as np
import jax
import jax.numpy as jnp
from jax.experimental import pallas as pl

L = 2048
S = 2720
P = 3072
D = 768
H = 12
DK = 64
EPS = 1e-5
BW = 144
KW0 = BW + 32
KWU = BW + 32 + 512


def _pe_np():
    position = np.arange(L, dtype=np.float32)[:, None]
    div = np.exp(np.arange(0, D, 2, dtype=np.float32) * -(math.log(10000.0) / D))
    pe = np.zeros((L, D), dtype=np.float32)
    pe[:, 0::2] = np.sin(position * div)
    pe[:, 1::2] = np.cos(position * div)
    return pe


_PE = _pe_np()


def _elu(x):
    return jnp.where(x > 0, x, jnp.exp(jnp.minimum(x, 0.0)) - 1.0)


def _ln_rows(x):
    m = jnp.mean(x, axis=-1, keepdims=True)
    v = jnp.mean((x - m) * (x - m), axis=-1, keepdims=True)
    return (x - m) * jax.lax.rsqrt(v + EPS)


def _embed_kernel(xemb_ref, wemb_ref, bemb_ref, pe_ref, dw_ref, db_ref,
                  cw0_ref, cb0_ref, cw1_ref, cb1_ref, cw2_ref, cb2_ref,
                  uw_ref, ub_ref, out_ref):
    seq = jnp.dot(xemb_ref[...], wemb_ref[...], preferred_element_type=jnp.float32)
    seq = seq + bemb_ref[...] + pe_ref[...]
    tmp = jnp.dot(seq, dw_ref[...], preferred_element_type=jnp.float32) + db_ref[...]
    o1 = _elu(jnp.dot(tmp.reshape(512, 512), cw0_ref[...],
                            preferred_element_type=jnp.float32) + cb0_ref[...])
    o2 = _elu(jnp.dot(o1.reshape(128, 512), cw1_ref[...],
                            preferred_element_type=jnp.float32) + cb1_ref[...])
    o3 = _elu(jnp.dot(o2.reshape(32, 512), cw2_ref[...],
                            preferred_element_type=jnp.float32) + cb2_ref[...])
    allin = jnp.concatenate([o1, o2, o3], axis=0)
    allin = jnp.dot(allin, uw_ref[...], preferred_element_type=jnp.float32) + ub_ref[...]
    full = jnp.concatenate([seq, allin], axis=0)
    out_ref[0:S, :] = _ln_rows(full)
    out_ref[S:P, :] = jnp.zeros((P - S, D), jnp.float32)


def _qkv_kernel(x_ref, wq_ref, wk_ref, wv_ref, bq_ref, bk_ref, bv_ref,
                q_ref, k_ref, v_ref):
    x = x_ref[...].astype(jnp.bfloat16)
    q = jnp.dot(x, wq_ref[...], preferred_element_type=jnp.float32) + bq_ref[...]
    q_ref[...] = (q * (1.0 / 8.0)).astype(jnp.bfloat16)
    k_ref[...] = (jnp.dot(x, wk_ref[...], preferred_element_type=jnp.float32)
                  + bk_ref[...]).astype(jnp.bfloat16)
    v_ref[...] = (jnp.dot(x, wv_ref[...], preferred_element_type=jnp.float32)
                  + bv_ref[...]).astype(jnp.bfloat16)


def _post_kernel(a_ref, res_ref, wo_ref, bo_ref, out_ref):
    y = jnp.dot(a_ref[...].astype(jnp.bfloat16), wo_ref[...],
                preferred_element_type=jnp.float32)
    out_ref[...] = _ln_rows(y + bo_ref[...] + res_ref[...])


def _ffn_kernel(x_ref, w1_ref, b1_ref, w2_ref, b2_ref, out_ref):
    x = x_ref[...]
    h = jax.nn.gelu(jnp.dot(x.astype(jnp.bfloat16), w1_ref[...],
                            preferred_element_type=jnp.float32) + b1_ref[...])
    y = jnp.dot(h.astype(jnp.bfloat16), w2_ref[...],
                preferred_element_type=jnp.float32) + b2_ref[...] + x
    out_ref[...] = _ln_rows(y)


def _attn_heads(q, kk, vv, ok, o_ref):
    for h in range(H):
        sl = slice(h * DK, (h + 1) * DK)
        s = jax.lax.dot_general(q[:, sl], kk[:, sl], (((1,), (1,)), ((), ())),
                                preferred_element_type=jnp.float32)
        s = jnp.where(ok, s, -1e9)
        m = jnp.max(s, axis=-1, keepdims=True)
        e = jnp.exp(s - m)
        p = (e / jnp.sum(e, axis=-1, keepdims=True)).astype(jnp.bfloat16)
        o_ref[:, sl] = jnp.dot(p, vv[:, sl], preferred_element_type=jnp.float32)


def _attn_l0_kernel(q_ref, k_ref, v_ref, o_ref):
    i = pl.program_id(0)
    qs = i * 128
    bs = pl.multiple_of(jnp.maximum(qs - 8, 0), 8)
    ps = pl.multiple_of(2048 + i * 32, 8)
    kk = jnp.concatenate([k_ref[pl.ds(bs, BW), :], k_ref[pl.ds(ps, 32), :]], axis=0)
    vv = jnp.concatenate([v_ref[pl.ds(bs, BW), :], v_ref[pl.ds(ps, 32), :]], axis=0)
    row = jax.lax.broadcasted_iota(jnp.int32, (128, KW0), 0)
    col = jax.lax.broadcasted_iota(jnp.int32, (128, KW0), 1)
    qi = qs + row
    absj = bs + col
    band_ok = (jnp.abs(absj - qi) <= 2) & (absj < 2048) & (col < BW)
    par_ok = (col >= BW) & (col - BW == row // 4)
    _attn_heads(q_ref[...], kk, vv, band_ok | par_ok, o_ref)


def _attn_up_kernel(q_ref, k_ref, v_ref, o_ref):
    i = pl.program_id(0)
    qs = 2048 + 128 * i
    ls = jnp.where(i < 4, 2048, jnp.where(i < 5, 2560, 2688))
    le = jnp.where(i < 4, 2560, jnp.where(i < 5, 2688, 2720))
    ps = jnp.where(i < 4, 2560 + 32 * i, 2688)
    hasp = i < 5
    cs = jnp.where(i < 4, 512 * i, jnp.where(i < 5, 2048, 2560))
    clen = jnp.where(i < 5, 512, 128)
    bs = pl.multiple_of(qs - 8, 8)
    ps = pl.multiple_of(ps, 8)
    cs = pl.multiple_of(cs, 8)
    kk = jnp.concatenate([k_ref[pl.ds(bs, BW), :], k_ref[pl.ds(ps, 32), :],
                          k_ref[pl.ds(cs, 512), :]], axis=0)
    vv = jnp.concatenate([v_ref[pl.ds(bs, BW), :], v_ref[pl.ds(ps, 32), :],
                          v_ref[pl.ds(cs, 512), :]], axis=0)
    row = jax.lax.broadcasted_iota(jnp.int32, (128, KWU), 0)
    col = jax.lax.broadcasted_iota(jnp.int32, (128, KWU), 1)
    qi = qs + row
    absj = bs + col
    band_ok = (jnp.abs(absj - qi) <= 2) & (absj >= ls) & (absj < le) & (col < BW)
    pj = col - BW
    par_ok = (col >= BW) & (col < BW + 32) & (pj == row // 4) & hasp
    cj = col - (BW + 32)
    chl_ok = (col >= BW + 32) & (cj >= 4 * row) & (cj < 4 * row + 4) & (cj < clen)
    _attn_heads(q_ref[...], kk, vv, band_ok | par_ok | chl_ok, o_ref)


def _gather_kernel(s0_ref, s1_ref, s2_ref, s3_ref, out_ref):
    out_ref[:, 0 * D:1 * D] = s0_ref[...]
    out_ref[:, 1 * D:2 * D] = jnp.repeat(s1_ref[...], 4, axis=0)
    out_ref[:, 2 * D:3 * D] = jnp.repeat(s2_ref[...], 16, axis=0)
    out_ref[:, 3 * D:4 * D] = jnp.repeat(s3_ref[...], 64, axis=0)


def _full_spec():
    return pl.BlockSpec((P, D), lambda i: (0, 0))


def kernel(x_enc, x_mark_enc, conv_w, conv_b, w_temp, b_temp, down_w, down_b,
           convs_w, convs_b, up_w, up_b, wq, bq, wk, bk, wv, bv, wo, bo,
           w1, b1, w2, b2):
    f32 = jnp.float32
    x = x_enc[0]
    xm = x_mark_enc[0]
    xcat = jnp.concatenate(
        [jnp.roll(x, 1, axis=0), x, jnp.roll(x, -1, axis=0), xm], axis=1)
    xemb = jnp.pad(xcat, ((0, 0), (0, 128 - 25)))
    wemb = jnp.pad(jnp.concatenate([conv_w.reshape(21, D), w_temp], axis=0),
                   ((0, 128 - 25), (0, 0)))
    bemb = (conv_b + b_temp)[None]
    pe = jnp.asarray(_PE)
    cw = [convs_w[i].reshape(512, 128) for i in range(3)]
    cb = [convs_b[i][None] for i in range(3)]

    seq = pl.pallas_call(
        _embed_kernel,
        out_shape=jax.ShapeDtypeStruct((P, D), f32),
    )(xemb, wemb, bemb, pe, down_w, down_b[None],
      cw[0], cb[0], cw[1], cb[1], cw[2], cb[2], up_w, up_b[None])

    row_spec = pl.BlockSpec((512, D), lambda i: (i, 0))
    w_spec = pl.BlockSpec((D, D), lambda i: (0, 0))
    b_spec = pl.BlockSpec((1, D), lambda i: (0, 0))

    for l in range(2):
        bf16 = jnp.bfloat16
        q, k, v = pl.pallas_call(
            _qkv_kernel,
            grid=(P // 512,),
            in_specs=[row_spec, w_spec, w_spec, w_spec, b_spec, b_spec, b_spec],
            out_specs=[row_spec, row_spec, row_spec],
            out_shape=[jax.ShapeDtypeStruct((P, D), bf16)] * 3,
        )(seq, wq[l].astype(bf16), wk[l].astype(bf16), wv[l].astype(bf16),
          bq[l][None], bk[l][None], bv[l][None])

        qblk = pl.BlockSpec((128, D), lambda i: (i, 0))
        o0 = pl.pallas_call(
            _attn_l0_kernel,
            grid=(16,),
            in_specs=[qblk, _full_spec(), _full_spec()],
            out_specs=qblk,
            out_shape=jax.ShapeDtypeStruct((2048, D), f32),
        )(q[:2048], k, v)
        qup = pl.BlockSpec((128, D), lambda i: (16 + i, 0))
        o1 = pl.pallas_call(
            _attn_up_kernel,
            grid=(6,),
            in_specs=[qup, _full_spec(), _full_spec()],
            out_specs=pl.BlockSpec((128, D), lambda i: (i, 0)),
            out_shape=jax.ShapeDtypeStruct((768, D), f32),
        )(q, k, v)
        attn = jnp.concatenate([o0, o1, jnp.zeros((P - 2816, D), f32)], axis=0)

        seq = pl.pallas_call(
            _post_kernel,
            grid=(P // 512,),
            in_specs=[row_spec, row_spec, w_spec, b_spec],
            out_specs=row_spec,
            out_shape=jax.ShapeDtypeStruct((P, D), f32),
        )(attn, seq, wo[l].astype(bf16), bo[l][None])

        seq = pl.pallas_call(
            _ffn_kernel,
            grid=(P // 512,),
            in_specs=[row_spec,
                      pl.BlockSpec((D, 2048), lambda i: (0, 0)),
                      pl.BlockSpec((1, 2048), lambda i: (0, 0)),
                      pl.BlockSpec((2048, D), lambda i: (0, 0)),
                      b_spec],
            out_specs=row_spec,
            out_shape=jax.ShapeDtypeStruct((P, D), f32),
        )(seq, w1[l].astype(bf16), b1[l][None], w2[l].astype(bf16), b2[l][None])

    out = pl.pallas_call(
        _gather_kernel,
        grid=(4,),
        in_specs=[pl.BlockSpec((512, D), lambda i: (i, 0)),
                  pl.BlockSpec((128, D), lambda i: (16 + i, 0)),
                  pl.BlockSpec((32, D), lambda i: (80 + i, 0)),
                  pl.BlockSpec((8, D), lambda i: (336 + i, 0))],
        out_specs=pl.BlockSpec((512, 4 * D), lambda i: (i, 0)),
        out_shape=jax.ShapeDtypeStruct((L, 4 * D), f32),
    )(seq, seq, seq, seq)
    return out[None]


# speedup vs baseline: 3.3738x; 1.3174x over previous
import functools
import math

import numpy as np
import jax
import jax.numpy as jnp
from jax.experimental import pallas as pl

L = 2048
S = 2720
P = 3072
D = 768
H = 12
DK = 64
EPS = 1e-5


def _pe_np():
    position = np.arange(L, dtype=np.float32)[:, None]
    div = np.exp(np.arange(0, D, 2, dtype=np.float32) * -(math.log(10000.0) / D))
    pe = np.zeros((L, D), dtype=np.float32)
    pe[:, 0::2] = np.sin(position * div)
    pe[:, 1::2] = np.cos(position * div)
    return pe


_PE = _pe_np()


def _elu(x):
    return jnp.where(x > 0, x, jnp.exp(jnp.minimum(x, 0.0)) - 1.0)


def _ln_rows(x):
    m = jnp.mean(x, axis=-1, keepdims=True)
    v = jnp.mean((x - m) * (x - m), axis=-1, keepdims=True)
    return (x - m) * jax.lax.rsqrt(v + EPS)


def _embed_kernel(xemb_ref, wemb_ref, bemb_ref, pe_ref, dw_ref, db_ref,
                  cw0_ref, cb0_ref, cw1_ref, cb1_ref, cw2_ref, cb2_ref,
                  uw_ref, ub_ref, out_ref):
    seq = jnp.dot(xemb_ref[...], wemb_ref[...], preferred_element_type=jnp.float32)
    seq = seq + bemb_ref[...] + pe_ref[...]
    tmp = jnp.dot(seq, dw_ref[...], preferred_element_type=jnp.float32) + db_ref[...]
    o1 = _elu(jnp.dot(tmp.reshape(512, 512), cw0_ref[...],
                      preferred_element_type=jnp.float32) + cb0_ref[...])
    o2 = _elu(jnp.dot(o1.reshape(128, 512), cw1_ref[...],
                      preferred_element_type=jnp.float32) + cb1_ref[...])
    o3 = _elu(jnp.dot(o2.reshape(32, 512), cw2_ref[...],
                      preferred_element_type=jnp.float32) + cb2_ref[...])
    allin = jnp.concatenate([o1, o2, o3], axis=0)
    allin = jnp.dot(allin, uw_ref[...], preferred_element_type=jnp.float32) + ub_ref[...]
    full = jnp.concatenate([seq, allin], axis=0)
    out_ref[0:S, :] = _ln_rows(full)
    out_ref[S:P, :] = jnp.zeros((P - S, D), jnp.float32)


def _qkv_kernel(x_ref, wq_ref, wk_ref, wv_ref, bq_ref, bk_ref, bv_ref,
                q_ref, k_ref, v_ref):
    x = x_ref[...].astype(jnp.bfloat16)
    q = jnp.dot(x, wq_ref[...], preferred_element_type=jnp.float32) + bq_ref[...]
    q_ref[...] = (q * (1.0 / 8.0)).astype(jnp.bfloat16)
    k_ref[...] = (jnp.dot(x, wk_ref[...], preferred_element_type=jnp.float32)
                  + bk_ref[...]).astype(jnp.bfloat16)
    v_ref[...] = (jnp.dot(x, wv_ref[...], preferred_element_type=jnp.float32)
                  + bv_ref[...]).astype(jnp.bfloat16)


def _post_kernel(a_ref, res_ref, wo_ref, bo_ref, out_ref):
    y = jnp.dot(a_ref[...].astype(jnp.bfloat16), wo_ref[...],
                preferred_element_type=jnp.float32)
    out_ref[...] = _ln_rows(y + bo_ref[...] + res_ref[...])


def _ffn_kernel(x_ref, w1_ref, b1_ref, w2_ref, b2_ref, out_ref):
    x = x_ref[...]
    h = jax.nn.gelu(jnp.dot(x.astype(jnp.bfloat16), w1_ref[...],
                            preferred_element_type=jnp.float32) + b1_ref[...])
    y = jnp.dot(h.astype(jnp.bfloat16), w2_ref[...],
                preferred_element_type=jnp.float32) + b2_ref[...] + x
    out_ref[...] = _ln_rows(y)


def _hsel(trans=False):
    f32 = jnp.float32
    if trans:
        a = jax.lax.broadcasted_iota(jnp.int32, (H, D), 0)
        b = jax.lax.broadcasted_iota(jnp.int32, (H, D), 1) // DK
    else:
        a = jax.lax.broadcasted_iota(jnp.int32, (D, H), 0) // DK
        b = jax.lax.broadcasted_iota(jnp.int32, (D, H), 1)
    return (a == b).astype(f32)


def _soft_av(svals, vlist, o_ref):
    f32 = jnp.float32
    et = _hsel(trans=True)
    m = functools.reduce(jnp.maximum, svals)
    es = [jnp.exp(s - m) for s in svals]
    z = functools.reduce(jnp.add, es)
    acc = None
    for e, vv in zip(es, vlist):
        p = jnp.dot(e / z, et, preferred_element_type=f32)
        acc = p * vv if acc is None else acc + p * vv
    o_ref[...] = acc


def _attn_l0_kernel(q_ref, k_ref, v_ref, o_ref):
    f32 = jnp.float32
    i = pl.program_id(0)
    qs = pl.multiple_of(i * 256, 8)
    ps = pl.multiple_of(2048 + 64 * i + 8, 8)
    q = q_ref[...].astype(f32)
    kb = k_ref[pl.ds(qs, 272), :]
    vb = v_ref[pl.ds(qs, 272), :]
    kp4 = jnp.repeat(k_ref[pl.ds(ps, 64), :], 4, axis=0).astype(f32)
    vp4 = jnp.repeat(v_ref[pl.ds(ps, 64), :], 4, axis=0).astype(f32)
    e_ = _hsel()
    rows = jax.lax.broadcasted_iota(jnp.int32, (256, 1), 0) + qs
    svals, vlist = [], []
    for d in range(-2, 3):
        kd = kb[8 + d:264 + d, :].astype(f32)
        s = jnp.dot(q * kd, e_, preferred_element_type=f32)
        ok = (rows + d >= 0) & (rows + d < 2048)
        svals.append(jnp.where(ok, s, -1e9))
        vlist.append(vb[8 + d:264 + d, :].astype(f32))
    svals.append(jnp.dot(q * kp4, e_, preferred_element_type=f32))
    vlist.append(vp4)
    _soft_av(svals, vlist, o_ref)


def _attn_up_kernel(q_ref, k_ref, v_ref, o_ref):
    f32 = jnp.float32
    i = pl.program_id(0)
    qs = 2048 + 128 * i
    ls = jnp.where(i < 4, 2048, jnp.where(i < 5, 2560, 2688))
    le = jnp.where(i < 4, 2560, jnp.where(i < 5, 2688, 2720))
    ps = jnp.where(i < 4, 2560 + 32 * i, 2688)
    hasp = i < 5
    cs = jnp.where(i < 4, 512 * i, jnp.where(i < 5, 2048, 2560))
    clen = jnp.where(i < 5, 512, 128)
    qsb = pl.multiple_of(qs, 8)
    psb = pl.multiple_of(ps + 8, 8)
    csb = pl.multiple_of(cs + 8, 8)
    q = q_ref[...].astype(f32)
    kb = k_ref[pl.ds(qsb, 144), :]
    vb = v_ref[pl.ds(qsb, 144), :]
    kp4 = jnp.repeat(k_ref[pl.ds(psb, 32), :], 4, axis=0).astype(f32)
    vp4 = jnp.repeat(v_ref[pl.ds(psb, 32), :], 4, axis=0).astype(f32)
    kc4 = k_ref[pl.ds(csb, 512), :].reshape(128, 4 * D)
    vc4 = v_ref[pl.ds(csb, 512), :].reshape(128, 4 * D)
    e_ = _hsel()
    rows = jax.lax.broadcasted_iota(jnp.int32, (128, 1), 0) + qs
    rloc = jax.lax.broadcasted_iota(jnp.int32, (128, 1), 0)
    svals, vlist = [], []
    for d in range(-2, 3):
        kd = kb[8 + d:136 + d, :].astype(f32)
        s = jnp.dot(q * kd, e_, preferred_element_type=f32)
        ok = (rows + d >= ls) & (rows + d < le)
        svals.append(jnp.where(ok, s, -1e9))
        vlist.append(vb[8 + d:136 + d, :].astype(f32))
    sp = jnp.dot(q * kp4, e_, preferred_element_type=f32)
    svals.append(jnp.where(hasp, sp, -1e9))
    vlist.append(vp4)
    for r in range(4):
        kcr = kc4[:, r * D:(r + 1) * D].astype(f32)
        s = jnp.dot(q * kcr, e_, preferred_element_type=f32)
        ok = 4 * rloc + r < clen
        svals.append(jnp.where(ok, s, -1e9))
        vlist.append(vc4[:, r * D:(r + 1) * D].astype(f32))
    _soft_av(svals, vlist, o_ref)


def _gather_kernel(s0_ref, s1_ref, s2_ref, s3_ref, out_ref):
    out_ref[:, 0 * D:1 * D] = s0_ref[...]
    out_ref[:, 1 * D:2 * D] = jnp.repeat(s1_ref[...], 4, axis=0)
    out_ref[:, 2 * D:3 * D] = jnp.repeat(s2_ref[...], 16, axis=0)
    out_ref[:, 3 * D:4 * D] = jnp.repeat(s3_ref[...], 64, axis=0)


def kernel(x_enc, x_mark_enc, conv_w, conv_b, w_temp, b_temp, down_w, down_b,
           convs_w, convs_b, up_w, up_b, wq, bq, wk, bk, wv, bv, wo, bo,
           w1, b1, w2, b2):
    f32 = jnp.float32
    bf16 = jnp.bfloat16
    x = x_enc[0]
    xm = x_mark_enc[0]
    xcat = jnp.concatenate(
        [jnp.roll(x, 1, axis=0), x, jnp.roll(x, -1, axis=0), xm], axis=1)
    xemb = jnp.pad(xcat, ((0, 0), (0, 128 - 25)))
    wemb = jnp.pad(jnp.concatenate([conv_w.reshape(21, D), w_temp], axis=0),
                   ((0, 128 - 25), (0, 0)))
    bemb = (conv_b + b_temp)[None]
    pe = jnp.asarray(_PE)
    cw = [convs_w[i].reshape(512, 128) for i in range(3)]
    cb = [convs_b[i][None] for i in range(3)]

    seq = pl.pallas_call(
        _embed_kernel,
        out_shape=jax.ShapeDtypeStruct((P, D), f32),
    )(xemb, wemb, bemb, pe, down_w, down_b[None],
      cw[0], cb[0], cw[1], cb[1], cw[2], cb[2], up_w, up_b[None])

    row_spec = pl.BlockSpec((512, D), lambda i: (i, 0))
    w_spec = pl.BlockSpec((D, D), lambda i: (0, 0))
    b_spec = pl.BlockSpec((1, D), lambda i: (0, 0))
    full_kv = pl.BlockSpec((P + 8, D), lambda i: (0, 0))

    for l in range(2):
        q, k, v = pl.pallas_call(
            _qkv_kernel,
            grid=(P // 512,),
            in_specs=[row_spec, w_spec, w_spec, w_spec, b_spec, b_spec, b_spec],
            out_specs=[row_spec, row_spec, row_spec],
            out_shape=[jax.ShapeDtypeStruct((P, D), bf16)] * 3,
        )(seq, wq[l].astype(bf16), wk[l].astype(bf16), wv[l].astype(bf16),
          bq[l][None], bk[l][None], bv[l][None])
        zero8 = jnp.zeros((8, D), bf16)
        k2 = jnp.concatenate([zero8, k], axis=0)
        v2 = jnp.concatenate([zero8, v], axis=0)

        o0 = pl.pallas_call(
            _attn_l0_kernel,
            grid=(8,),
            in_specs=[pl.BlockSpec((256, D), lambda i: (i, 0)), full_kv, full_kv],
            out_specs=pl.BlockSpec((256, D), lambda i: (i, 0)),
            out_shape=jax.ShapeDtypeStruct((2048, D), f32),
        )(q[:2048], k2, v2)
        o1 = pl.pallas_call(
            _attn_up_kernel,
            grid=(6,),
            in_specs=[pl.BlockSpec((128, D), lambda i: (16 + i, 0)), full_kv, full_kv],
            out_specs=pl.BlockSpec((128, D), lambda i: (i, 0)),
            out_shape=jax.ShapeDtypeStruct((768, D), f32),
        )(q, k2, v2)
        attn = jnp.concatenate([o0, o1, jnp.zeros((P - 2816, D), f32)], axis=0)

        seq = pl.pallas_call(
            _post_kernel,
            grid=(P // 512,),
            in_specs=[row_spec, row_spec, w_spec, b_spec],
            out_specs=row_spec,
            out_shape=jax.ShapeDtypeStruct((P, D), f32),
        )(attn, seq, wo[l].astype(bf16), bo[l][None])

        seq = pl.pallas_call(
            _ffn_kernel,
            grid=(P // 512,),
            in_specs=[row_spec,
                      pl.BlockSpec((D, 2048), lambda i: (0, 0)),
                      pl.BlockSpec((1, 2048), lambda i: (0, 0)),
                      pl.BlockSpec((2048, D), lambda i: (0, 0)),
                      b_spec],
            out_specs=row_spec,
            out_shape=jax.ShapeDtypeStruct((P, D), f32),
        )(seq, w1[l].astype(bf16), b1[l][None], w2[l].astype(bf16), b2[l][None])

    out = pl.pallas_call(
        _gather_kernel,
        grid=(4,),
        in_specs=[pl.BlockSpec((512, D), lambda i: (i, 0)),
                  pl.BlockSpec((128, D), lambda i: (16 + i, 0)),
                  pl.BlockSpec((32, D), lambda i: (80 + i, 0)),
                  pl.BlockSpec((8, D), lambda i: (336 + i, 0))],
        out_specs=pl.BlockSpec((512, 4 * D), lambda i: (i, 0)),
        out_shape=jax.ShapeDtypeStruct((L, 4 * D), f32),
    )(seq, seq, seq, seq)
    return out[None]


# speedup vs baseline: 3.4951x; 1.0360x over previous
import functools
import math

import numpy as np
import jax
import jax.numpy as jnp
from jax.experimental import pallas as pl

L = 2048
S = 2720
P = 3072
D = 768
H = 12
DK = 64
EPS = 1e-5


def _pe_np():
    position = np.arange(L, dtype=np.float32)[:, None]
    div = np.exp(np.arange(0, D, 2, dtype=np.float32) * -(math.log(10000.0) / D))
    pe = np.zeros((L, D), dtype=np.float32)
    pe[:, 0::2] = np.sin(position * div)
    pe[:, 1::2] = np.cos(position * div)
    return pe


_PE = _pe_np()


def _elu(x):
    return jnp.where(x > 0, x, jnp.exp(jnp.minimum(x, 0.0)) - 1.0)


def _ln_rows(x):
    m = jnp.mean(x, axis=-1, keepdims=True)
    v = jnp.mean((x - m) * (x - m), axis=-1, keepdims=True)
    return (x - m) * jax.lax.rsqrt(v + EPS)


def _embed_kernel(xemb_ref, wemb_ref, bemb_ref, pe_ref, dw_ref, db_ref,
                  cw0_ref, cb0_ref, cw1_ref, cb1_ref, cw2_ref, cb2_ref,
                  uw_ref, ub_ref, out_ref):
    seq = jnp.dot(xemb_ref[...], wemb_ref[...], preferred_element_type=jnp.float32)
    seq = seq + bemb_ref[...] + pe_ref[...]
    tmp = jnp.dot(seq, dw_ref[...], preferred_element_type=jnp.float32) + db_ref[...]
    o1 = _elu(jnp.dot(tmp.reshape(512, 512), cw0_ref[...],
                      preferred_element_type=jnp.float32) + cb0_ref[...])
    o2 = _elu(jnp.dot(o1.reshape(128, 512), cw1_ref[...],
                      preferred_element_type=jnp.float32) + cb1_ref[...])
    o3 = _elu(jnp.dot(o2.reshape(32, 512), cw2_ref[...],
                      preferred_element_type=jnp.float32) + cb2_ref[...])
    allin = jnp.concatenate([o1, o2, o3], axis=0)
    allin = jnp.dot(allin, uw_ref[...], preferred_element_type=jnp.float32) + ub_ref[...]
    full = jnp.concatenate([seq, allin], axis=0)
    out_ref[0:S, :] = _ln_rows(full)
    out_ref[S:P, :] = jnp.zeros((P - S, D), jnp.float32)


def _qkv_kernel(x_ref, wq_ref, wk_ref, wv_ref, bq_ref, bk_ref, bv_ref,
                q_ref, k_ref, v_ref):
    x = x_ref[...].astype(jnp.bfloat16)
    q = jnp.dot(x, wq_ref[...], preferred_element_type=jnp.float32) + bq_ref[...]
    q_ref[...] = (q * (1.0 / 8.0)).astype(jnp.bfloat16)
    k_ref[...] = (jnp.dot(x, wk_ref[...], preferred_element_type=jnp.float32)
                  + bk_ref[...]).astype(jnp.bfloat16)
    v_ref[...] = (jnp.dot(x, wv_ref[...], preferred_element_type=jnp.float32)
                  + bv_ref[...]).astype(jnp.bfloat16)


def _ffn_kernel(x_ref, w1_ref, b1_ref, w2_ref, b2_ref, out_ref):
    x = x_ref[...]
    h = jax.nn.gelu(jnp.dot(x.astype(jnp.bfloat16), w1_ref[...],
                            preferred_element_type=jnp.float32) + b1_ref[...])
    y = jnp.dot(h.astype(jnp.bfloat16), w2_ref[...],
                preferred_element_type=jnp.float32) + b2_ref[...] + x
    out_ref[...] = _ln_rows(y)


def _hsel(trans=False):
    f32 = jnp.float32
    if trans:
        a = jax.lax.broadcasted_iota(jnp.int32, (H, D), 0)
        b = jax.lax.broadcasted_iota(jnp.int32, (H, D), 1) // DK
    else:
        a = jax.lax.broadcasted_iota(jnp.int32, (D, H), 0) // DK
        b = jax.lax.broadcasted_iota(jnp.int32, (D, H), 1)
    return (a == b).astype(f32)


def _soft_av(svals, vlist):
    f32 = jnp.float32
    et = _hsel(trans=True)
    m = functools.reduce(jnp.maximum, svals)
    es = [jnp.exp(s - m) for s in svals]
    z = functools.reduce(jnp.add, es)
    acc = None
    for e, vv in zip(es, vlist):
        p = jnp.dot(e / z, et, preferred_element_type=f32)
        acc = p * vv if acc is None else acc + p * vv
    return acc


def _post_ln(acc, res_ref, wo_ref, bo_ref, o_ref):
    y = jnp.dot(acc.astype(jnp.bfloat16), wo_ref[...],
                preferred_element_type=jnp.float32)
    o_ref[...] = _ln_rows(y + bo_ref[...] + res_ref[...])


def _attn_l0_kernel(q_ref, k_ref, v_ref, res_ref, wo_ref, bo_ref, o_ref):
    f32 = jnp.float32
    i = pl.program_id(0)
    qs = pl.multiple_of(i * 256, 8)
    ps = pl.multiple_of(2048 + 64 * i + 8, 8)
    q = q_ref[...].astype(f32)
    kb = k_ref[pl.ds(qs, 272), :]
    vb = v_ref[pl.ds(qs, 272), :]
    kp4 = jnp.repeat(k_ref[pl.ds(ps, 64), :], 4, axis=0).astype(f32)
    vp4 = jnp.repeat(v_ref[pl.ds(ps, 64), :], 4, axis=0).astype(f32)
    e_ = _hsel()
    rows = jax.lax.broadcasted_iota(jnp.int32, (256, 1), 0) + qs
    svals, vlist = [], []
    for d in range(-2, 3):
        kd = kb[8 + d:264 + d, :].astype(f32)
        s = jnp.dot(q * kd, e_, preferred_element_type=f32)
        ok = (rows + d >= 0) & (rows + d < 2048)
        svals.append(jnp.where(ok, s, -1e9))
        vlist.append(vb[8 + d:264 + d, :].astype(f32))
    svals.append(jnp.dot(q * kp4, e_, preferred_element_type=f32))
    vlist.append(vp4)
    _post_ln(_soft_av(svals, vlist), res_ref, wo_ref, bo_ref, o_ref)


def _attn_up_kernel(q_ref, k_ref, v_ref, res_ref, wo_ref, bo_ref, o_ref):
    f32 = jnp.float32
    i = pl.program_id(0)
    qs = 2048 + 128 * i
    ls = jnp.where(i < 4, 2048, jnp.where(i < 5, 2560, 2688))
    le = jnp.where(i < 4, 2560, jnp.where(i < 5, 2688, 2720))
    ps = jnp.where(i < 4, 2560 + 32 * i, 2688)
    hasp = i < 5
    cs = jnp.where(i < 4, 512 * i, jnp.where(i < 5, 2048, 2560))
    clen = jnp.where(i < 5, 512, 128)
    qsb = pl.multiple_of(qs, 8)
    psb = pl.multiple_of(ps + 8, 8)
    csb = pl.multiple_of(cs + 8, 8)
    q = q_ref[...].astype(f32)
    kb = k_ref[pl.ds(qsb, 144), :]
    vb = v_ref[pl.ds(qsb, 144), :]
    kp4 = jnp.repeat(k_ref[pl.ds(psb, 32), :], 4, axis=0).astype(f32)
    vp4 = jnp.repeat(v_ref[pl.ds(psb, 32), :], 4, axis=0).astype(f32)
    kc4 = k_ref[pl.ds(csb, 512), :].reshape(128, 4 * D)
    vc4 = v_ref[pl.ds(csb, 512), :].reshape(128, 4 * D)
    e_ = _hsel()
    rows = jax.lax.broadcasted_iota(jnp.int32, (128, 1), 0) + qs
    rloc = jax.lax.broadcasted_iota(jnp.int32, (128, 1), 0)
    svals, vlist = [], []
    for d in range(-2, 3):
        kd = kb[8 + d:136 + d, :].astype(f32)
        s = jnp.dot(q * kd, e_, preferred_element_type=f32)
        ok = (rows + d >= ls) & (rows + d < le)
        svals.append(jnp.where(ok, s, -1e9))
        vlist.append(vb[8 + d:136 + d, :].astype(f32))
    sp = jnp.dot(q * kp4, e_, preferred_element_type=f32)
    svals.append(jnp.where(hasp, sp, -1e9))
    vlist.append(vp4)
    for r in range(4):
        kcr = kc4[:, r * D:(r + 1) * D].astype(f32)
        s = jnp.dot(q * kcr, e_, preferred_element_type=f32)
        ok = 4 * rloc + r < clen
        svals.append(jnp.where(ok, s, -1e9))
        vlist.append(vc4[:, r * D:(r + 1) * D].astype(f32))
    _post_ln(_soft_av(svals, vlist), res_ref, wo_ref, bo_ref, o_ref)


def _gather_kernel(s0_ref, s1_ref, s2_ref, s3_ref, out_ref):
    out_ref[:, 0 * D:1 * D] = s0_ref[...]
    out_ref[:, 1 * D:2 * D] = jnp.repeat(s1_ref[...], 4, axis=0)
    out_ref[:, 2 * D:3 * D] = jnp.repeat(s2_ref[...], 16, axis=0)
    out_ref[:, 3 * D:4 * D] = jnp.repeat(s3_ref[...], 64, axis=0)


def kernel(x_enc, x_mark_enc, conv_w, conv_b, w_temp, b_temp, down_w, down_b,
           convs_w, convs_b, up_w, up_b, wq, bq, wk, bk, wv, bv, wo, bo,
           w1, b1, w2, b2):
    f32 = jnp.float32
    bf16 = jnp.bfloat16
    x = x_enc[0]
    xm = x_mark_enc[0]
    xcat = jnp.concatenate(
        [jnp.roll(x, 1, axis=0), x, jnp.roll(x, -1, axis=0), xm], axis=1)
    xemb = jnp.pad(xcat, ((0, 0), (0, 128 - 25)))
    wemb = jnp.pad(jnp.concatenate([conv_w.reshape(21, D), w_temp], axis=0),
                   ((0, 128 - 25), (0, 0)))
    bemb = (conv_b + b_temp)[None]
    pe = jnp.asarray(_PE)
    cw = [convs_w[i].reshape(512, 128) for i in range(3)]
    cb = [convs_b[i][None] for i in range(3)]

    seq = pl.pallas_call(
        _embed_kernel,
        out_shape=jax.ShapeDtypeStruct((P, D), f32),
    )(xemb, wemb, bemb, pe, down_w, down_b[None],
      cw[0], cb[0], cw[1], cb[1], cw[2], cb[2], up_w, up_b[None])

    row_spec = pl.BlockSpec((512, D), lambda i: (i, 0))
    w_spec = pl.BlockSpec((D, D), lambda i: (0, 0))
    b_spec = pl.BlockSpec((1, D), lambda i: (0, 0))
    full_kv = pl.BlockSpec((P + 8, D), lambda i: (0, 0))

    for l in range(2):
        q, k, v = pl.pallas_call(
            _qkv_kernel,
            grid=(P // 512,),
            in_specs=[row_spec, w_spec, w_spec, w_spec, b_spec, b_spec, b_spec],
            out_specs=[row_spec, row_spec, row_spec],
            out_shape=[jax.ShapeDtypeStruct((P, D), bf16)] * 3,
        )(seq, wq[l].astype(bf16), wk[l].astype(bf16), wv[l].astype(bf16),
          bq[l][None], bk[l][None], bv[l][None])
        zero8 = jnp.zeros((8, D), bf16)
        k2 = jnp.concatenate([zero8, k], axis=0)
        v2 = jnp.concatenate([zero8, v], axis=0)

        wo_b = wo[l].astype(bf16)
        bo_b = bo[l][None]
        o0 = pl.pallas_call(
            _attn_l0_kernel,
            grid=(8,),
            in_specs=[pl.BlockSpec((256, D), lambda i: (i, 0)), full_kv, full_kv,
                      pl.BlockSpec((256, D), lambda i: (i, 0)), w_spec, b_spec],
            out_specs=pl.BlockSpec((256, D), lambda i: (i, 0)),
            out_shape=jax.ShapeDtypeStruct((2048, D), f32),
        )(q[:2048], k2, v2, seq, wo_b, bo_b)
        o1 = pl.pallas_call(
            _attn_up_kernel,
            grid=(6,),
            in_specs=[pl.BlockSpec((128, D), lambda i: (16 + i, 0)), full_kv, full_kv,
                      pl.BlockSpec((128, D), lambda i: (16 + i, 0)), w_spec, b_spec],
            out_specs=pl.BlockSpec((128, D), lambda i: (i, 0)),
            out_shape=jax.ShapeDtypeStruct((768, D), f32),
        )(q, k2, v2, seq, wo_b, bo_b)
        seq = jnp.concatenate([o0, o1, seq[2816:]], axis=0)

        seq = pl.pallas_call(
            _ffn_kernel,
            grid=(P // 512,),
            in_specs=[row_spec,
                      pl.BlockSpec((D, 2048), lambda i: (0, 0)),
                      pl.BlockSpec((1, 2048), lambda i: (0, 0)),
                      pl.BlockSpec((2048, D), lambda i: (0, 0)),
                      b_spec],
            out_specs=row_spec,
            out_shape=jax.ShapeDtypeStruct((P, D), f32),
        )(seq, w1[l].astype(bf16), b1[l][None], w2[l].astype(bf16), b2[l][None])

    out = pl.pallas_call(
        _gather_kernel,
        grid=(4,),
        in_specs=[pl.BlockSpec((512, D), lambda i: (i, 0)),
                  pl.BlockSpec((128, D), lambda i: (16 + i, 0)),
                  pl.BlockSpec((32, D), lambda i: (80 + i, 0)),
                  pl.BlockSpec((8, D), lambda i: (336 + i, 0))],
        out_specs=pl.BlockSpec((512, 4 * D), lambda i: (i, 0)),
        out_shape=jax.ShapeDtypeStruct((L, 4 * D), f32),
    )(seq, seq, seq, seq)
    return out[None]


# speedup vs baseline: 3.8580x; 1.1038x over previous
import functools
import math

import numpy as np
import jax
import jax.numpy as jnp
from jax.experimental import pallas as pl

L = 2048
S = 2720
P = 3072
D = 768
H = 12
DK = 64
EPS = 1e-5


def _pe_np():
    position = np.arange(L, dtype=np.float32)[:, None]
    div = np.exp(np.arange(0, D, 2, dtype=np.float32) * -(math.log(10000.0) / D))
    pe = np.zeros((L, D), dtype=np.float32)
    pe[:, 0::2] = np.sin(position * div)
    pe[:, 1::2] = np.cos(position * div)
    return pe


_PE = _pe_np()


def _elu(x):
    return jnp.where(x > 0, x, jnp.exp(jnp.minimum(x, 0.0)) - 1.0)


def _ln_rows(x):
    m = jnp.mean(x, axis=-1, keepdims=True)
    v = jnp.mean((x - m) * (x - m), axis=-1, keepdims=True)
    return (x - m) * jax.lax.rsqrt(v + EPS)


def _embed_kernel(xemb_ref, wemb_ref, bemb_ref, pe_ref, dw_ref, db_ref,
                  cw0_ref, cb0_ref, cw1_ref, cb1_ref, cw2_ref, cb2_ref,
                  uw_ref, ub_ref, wq_ref, wk_ref, wv_ref,
                  bq_ref, bk_ref, bv_ref, out_ref, q_ref, k_ref, v_ref):
    seq = jnp.dot(xemb_ref[...], wemb_ref[...], preferred_element_type=jnp.float32)
    seq = seq + bemb_ref[...] + pe_ref[...]
    tmp = jnp.dot(seq, dw_ref[...], preferred_element_type=jnp.float32) + db_ref[...]
    o1 = _elu(jnp.dot(tmp.reshape(512, 512), cw0_ref[...],
                      preferred_element_type=jnp.float32) + cb0_ref[...])
    o2 = _elu(jnp.dot(o1.reshape(128, 512), cw1_ref[...],
                      preferred_element_type=jnp.float32) + cb1_ref[...])
    o3 = _elu(jnp.dot(o2.reshape(32, 512), cw2_ref[...],
                      preferred_element_type=jnp.float32) + cb2_ref[...])
    allin = jnp.concatenate([o1, o2, o3], axis=0)
    allin = jnp.dot(allin, uw_ref[...], preferred_element_type=jnp.float32) + ub_ref[...]
    full = jnp.concatenate([seq, allin], axis=0)
    sfull = _ln_rows(full)
    out_ref[0:S, :] = sfull
    out_ref[S:P, :] = jnp.zeros((P - S, D), jnp.float32)
    bf16 = jnp.bfloat16
    sb = sfull.astype(bf16)
    qv = jnp.dot(sb, wq_ref[...], preferred_element_type=jnp.float32) + bq_ref[...]
    q_ref[0:S, :] = (qv * (1.0 / 8.0)).astype(bf16)
    q_ref[S:P, :] = jnp.zeros((P - S, D), bf16)
    k_ref[0:8, :] = jnp.zeros((8, D), bf16)
    k_ref[8:8 + S, :] = (jnp.dot(sb, wk_ref[...], preferred_element_type=jnp.float32)
                         + bk_ref[...]).astype(bf16)
    k_ref[8 + S:P + 8, :] = jnp.zeros((P - S, D), bf16)
    v_ref[0:8, :] = jnp.zeros((8, D), bf16)
    v_ref[8:8 + S, :] = (jnp.dot(sb, wv_ref[...], preferred_element_type=jnp.float32)
                         + bv_ref[...]).astype(bf16)
    v_ref[8 + S:P + 8, :] = jnp.zeros((P - S, D), bf16)


def _ffn_kernel(x_ref, w1_ref, b1_ref, w2_ref, b2_ref, out_ref):
    x = x_ref[...]
    h = jax.nn.gelu(jnp.dot(x.astype(jnp.bfloat16), w1_ref[...],
                            preferred_element_type=jnp.float32) + b1_ref[...])
    y = jnp.dot(h.astype(jnp.bfloat16), w2_ref[...],
                preferred_element_type=jnp.float32) + b2_ref[...] + x
    out_ref[...] = _ln_rows(y)


def _ffn_qkv_kernel(x_ref, w1_ref, b1_ref, w2_ref, b2_ref,
                    wq_ref, wk_ref, wv_ref, bq_ref, bk_ref, bv_ref,
                    out_ref, q_ref, k_ref, v_ref):
    bf16 = jnp.bfloat16
    x = x_ref[...]
    h = jax.nn.gelu(jnp.dot(x.astype(bf16), w1_ref[...],
                            preferred_element_type=jnp.float32) + b1_ref[...])
    y = _ln_rows(jnp.dot(h.astype(bf16), w2_ref[...],
                         preferred_element_type=jnp.float32) + b2_ref[...] + x)
    out_ref[...] = y
    yb = y.astype(bf16)
    qv = jnp.dot(yb, wq_ref[...], preferred_element_type=jnp.float32) + bq_ref[...]
    q_ref[...] = (qv * (1.0 / 8.0)).astype(bf16)
    k_ref[...] = (jnp.dot(yb, wk_ref[...], preferred_element_type=jnp.float32)
                  + bk_ref[...]).astype(bf16)
    v_ref[...] = (jnp.dot(yb, wv_ref[...], preferred_element_type=jnp.float32)
                  + bv_ref[...]).astype(bf16)


def _hsel(trans=False):
    f32 = jnp.float32
    if trans:
        a = jax.lax.broadcasted_iota(jnp.int32, (H, D), 0)
        b = jax.lax.broadcasted_iota(jnp.int32, (H, D), 1) // DK
    else:
        a = jax.lax.broadcasted_iota(jnp.int32, (D, H), 0) // DK
        b = jax.lax.broadcasted_iota(jnp.int32, (D, H), 1)
    return (a == b).astype(f32)


def _soft_av(svals, vlist):
    f32 = jnp.float32
    et = _hsel(trans=True)
    m = functools.reduce(jnp.maximum, svals)
    es = [jnp.exp(s - m) for s in svals]
    z = functools.reduce(jnp.add, es)
    acc = None
    for e, vv in zip(es, vlist):
        p = jnp.dot(e / z, et, preferred_element_type=f32)
        acc = p * vv if acc is None else acc + p * vv
    return acc


def _post_ln(acc, res_ref, wo_ref, bo_ref, o_ref):
    y = jnp.dot(acc.astype(jnp.bfloat16), wo_ref[...],
                preferred_element_type=jnp.float32)
    o_ref[...] = _ln_rows(y + bo_ref[...] + res_ref[...])


def _attn_l0_kernel(q_ref, k_ref, v_ref, res_ref, wo_ref, bo_ref, o_ref):
    f32 = jnp.float32
    i = pl.program_id(0)
    qs = pl.multiple_of(i * 256, 8)
    ps = pl.multiple_of(2048 + 64 * i + 8, 8)
    q = q_ref[...].astype(f32)
    kb = k_ref[pl.ds(qs, 272), :]
    vb = v_ref[pl.ds(qs, 272), :]
    kp4 = jnp.repeat(k_ref[pl.ds(ps, 64), :], 4, axis=0).astype(f32)
    vp4 = jnp.repeat(v_ref[pl.ds(ps, 64), :], 4, axis=0).astype(f32)
    e_ = _hsel()
    rows = jax.lax.broadcasted_iota(jnp.int32, (256, 1), 0) + qs
    svals, vlist = [], []
    for d in range(-2, 3):
        kd = kb[8 + d:264 + d, :].astype(f32)
        s = jnp.dot(q * kd, e_, preferred_element_type=f32)
        ok = (rows + d >= 0) & (rows + d < 2048)
        svals.append(jnp.where(ok, s, -1e9))
        vlist.append(vb[8 + d:264 + d, :].astype(f32))
    svals.append(jnp.dot(q * kp4, e_, preferred_element_type=f32))
    vlist.append(vp4)
    _post_ln(_soft_av(svals, vlist), res_ref, wo_ref, bo_ref, o_ref)


def _attn_up_kernel(q_ref, k_ref, v_ref, res_ref, wo_ref, bo_ref, o_ref):
    f32 = jnp.float32
    i = pl.program_id(0)
    qs = 2048 + 128 * i
    ls = jnp.where(i < 4, 2048, jnp.where(i < 5, 2560, 2688))
    le = jnp.where(i < 4, 2560, jnp.where(i < 5, 2688, 2720))
    ps = jnp.where(i < 4, 2560 + 32 * i, 2688)
    hasp = i < 5
    cs = jnp.where(i < 4, 512 * i, jnp.where(i < 5, 2048, 2560))
    clen = jnp.where(i < 5, 512, 128)
    qsb = pl.multiple_of(qs, 8)
    psb = pl.multiple_of(ps + 8, 8)
    csb = pl.multiple_of(cs + 8, 8)
    q = q_ref[...].astype(f32)
    kb = k_ref[pl.ds(qsb, 144), :]
    vb = v_ref[pl.ds(qsb, 144), :]
    kp4 = jnp.repeat(k_ref[pl.ds(psb, 32), :], 4, axis=0).astype(f32)
    vp4 = jnp.repeat(v_ref[pl.ds(psb, 32), :], 4, axis=0).astype(f32)
    kc4 = k_ref[pl.ds(csb, 512), :].reshape(128, 4 * D)
    vc4 = v_ref[pl.ds(csb, 512), :].reshape(128, 4 * D)
    e_ = _hsel()
    rows = jax.lax.broadcasted_iota(jnp.int32, (128, 1), 0) + qs
    rloc = jax.lax.broadcasted_iota(jnp.int32, (128, 1), 0)
    svals, vlist = [], []
    for d in range(-2, 3):
        kd = kb[8 + d:136 + d, :].astype(f32)
        s = jnp.dot(q * kd, e_, preferred_element_type=f32)
        ok = (rows + d >= ls) & (rows + d < le)
        svals.append(jnp.where(ok, s, -1e9))
        vlist.append(vb[8 + d:136 + d, :].astype(f32))
    sp = jnp.dot(q * kp4, e_, preferred_element_type=f32)
    svals.append(jnp.where(hasp, sp, -1e9))
    vlist.append(vp4)
    for r in range(4):
        kcr = kc4[:, r * D:(r + 1) * D].astype(f32)
        s = jnp.dot(q * kcr, e_, preferred_element_type=f32)
        ok = 4 * rloc + r < clen
        svals.append(jnp.where(ok, s, -1e9))
        vlist.append(vc4[:, r * D:(r + 1) * D].astype(f32))
    _post_ln(_soft_av(svals, vlist), res_ref, wo_ref, bo_ref, o_ref)


def _gather_kernel(s0_ref, s1_ref, s2_ref, s3_ref, out_ref):
    out_ref[:, 0 * D:1 * D] = s0_ref[...]
    out_ref[:, 1 * D:2 * D] = jnp.repeat(s1_ref[...], 4, axis=0)
    out_ref[:, 2 * D:3 * D] = jnp.repeat(s2_ref[...], 16, axis=0)
    out_ref[:, 3 * D:4 * D] = jnp.repeat(s3_ref[...], 64, axis=0)


def kernel(x_enc, x_mark_enc, conv_w, conv_b, w_temp, b_temp, down_w, down_b,
           convs_w, convs_b, up_w, up_b, wq, bq, wk, bk, wv, bv, wo, bo,
           w1, b1, w2, b2):
    f32 = jnp.float32
    bf16 = jnp.bfloat16
    x = x_enc[0]
    xm = x_mark_enc[0]
    xcat = jnp.concatenate(
        [jnp.roll(x, 1, axis=0), x, jnp.roll(x, -1, axis=0), xm], axis=1)
    xemb = jnp.pad(xcat, ((0, 0), (0, 128 - 25)))
    wemb = jnp.pad(jnp.concatenate([conv_w.reshape(21, D), w_temp], axis=0),
                   ((0, 128 - 25), (0, 0)))
    bemb = (conv_b + b_temp)[None]
    pe = jnp.asarray(_PE)
    cw = [convs_w[i].reshape(512, 128) for i in range(3)]
    cb = [convs_b[i][None] for i in range(3)]

    row_spec = pl.BlockSpec((512, D), lambda i: (i, 0))
    w_spec = pl.BlockSpec((D, D), lambda i: (0, 0))
    b_spec = pl.BlockSpec((1, D), lambda i: (0, 0))
    full_kv = pl.BlockSpec((P + 8, D), lambda i: (0, 0))

    wqb = [wq[l].astype(bf16) for l in range(2)]
    wkb = [wk[l].astype(bf16) for l in range(2)]
    wvb = [wv[l].astype(bf16) for l in range(2)]
    wob = [wo[l].astype(bf16) for l in range(2)]

    seq, q, k2, v2 = pl.pallas_call(
        _embed_kernel,
        out_shape=[jax.ShapeDtypeStruct((P, D), f32),
                   jax.ShapeDtypeStruct((P, D), bf16),
                   jax.ShapeDtypeStruct((P + 8, D), bf16),
                   jax.ShapeDtypeStruct((P + 8, D), bf16)],
    )(xemb, wemb, bemb, pe, down_w, down_b[None],
      cw[0], cb[0], cw[1], cb[1], cw[2], cb[2], up_w, up_b[None],
      wqb[0], wkb[0], wvb[0], bq[0][None], bk[0][None], bv[0][None])

    for l in range(2):
        seq = pl.pallas_call(
            _attn_l0_kernel,
            grid=(8,),
            in_specs=[pl.BlockSpec((256, D), lambda i: (i, 0)), full_kv, full_kv,
                      pl.BlockSpec((256, D), lambda i: (i, 0)), w_spec, b_spec],
            out_specs=pl.BlockSpec((256, D), lambda i: (i, 0)),
            out_shape=jax.ShapeDtypeStruct((P, D), f32),
            input_output_aliases={3: 0},
        )(q, k2, v2, seq, wob[l], bo[l][None])
        seq = pl.pallas_call(
            _attn_up_kernel,
            grid=(6,),
            in_specs=[pl.BlockSpec((128, D), lambda i: (16 + i, 0)), full_kv, full_kv,
                      pl.BlockSpec((128, D), lambda i: (16 + i, 0)), w_spec, b_spec],
            out_specs=pl.BlockSpec((128, D), lambda i: (16 + i, 0)),
            out_shape=jax.ShapeDtypeStruct((P, D), f32),
            input_output_aliases={3: 0},
        )(q, k2, v2, seq, wob[l], bo[l][None])

        if l == 0:
            seq, q, k, v = pl.pallas_call(
                _ffn_qkv_kernel,
                grid=(P // 512,),
                in_specs=[row_spec,
                          pl.BlockSpec((D, 2048), lambda i: (0, 0)),
                          pl.BlockSpec((1, 2048), lambda i: (0, 0)),
                          pl.BlockSpec((2048, D), lambda i: (0, 0)),
                          b_spec, w_spec, w_spec, w_spec,
                          b_spec, b_spec, b_spec],
                out_specs=[row_spec, row_spec, row_spec, row_spec],
                out_shape=[jax.ShapeDtypeStruct((P, D), f32),
                           jax.ShapeDtypeStruct((P, D), bf16),
                           jax.ShapeDtypeStruct((P, D), bf16),
                           jax.ShapeDtypeStruct((P, D), bf16)],
            )(seq, w1[0].astype(bf16), b1[0][None], w2[0].astype(bf16), b2[0][None],
              wqb[1], wkb[1], wvb[1], bq[1][None], bk[1][None], bv[1][None])
            zero8 = jnp.zeros((8, D), bf16)
            k2 = jnp.concatenate([zero8, k], axis=0)
            v2 = jnp.concatenate([zero8, v], axis=0)
        else:
            seq = pl.pallas_call(
                _ffn_kernel,
                grid=(P // 512,),
                in_specs=[row_spec,
                          pl.BlockSpec((D, 2048), lambda i: (0, 0)),
                          pl.BlockSpec((1, 2048), lambda i: (0, 0)),
                          pl.BlockSpec((2048, D), lambda i: (0, 0)),
                          b_spec],
                out_specs=row_spec,
                out_shape=jax.ShapeDtypeStruct((P, D), f32),
            )(seq, w1[1].astype(bf16), b1[1][None], w2[1].astype(bf16), b2[1][None])

    out = pl.pallas_call(
        _gather_kernel,
        grid=(4,),
        in_specs=[pl.BlockSpec((512, D), lambda i: (i, 0)),
                  pl.BlockSpec((128, D), lambda i: (16 + i, 0)),
                  pl.BlockSpec((32, D), lambda i: (80 + i, 0)),
                  pl.BlockSpec((8, D), lambda i: (336 + i, 0))],
        out_specs=pl.BlockSpec((512, 4 * D), lambda i: (i, 0)),
        out_shape=jax.ShapeDtypeStruct((L, 4 * D), f32),
    )(seq, seq, seq, seq)
    return out[None]
